# baseline (speedup 1.0000x reference)
import functools
import math

import jax
import jax.numpy as jnp
import numpy as np
from jax.experimental import pallas as pl
from jax.experimental.pallas import tpu as pltpu

N_NODES = 10000
RATIO = 0.5

NP = 10240
K1, K2, K3 = 5000, 2500, 1250
K1P, K2P, K3P = 5120, 2560, 1280


def _pick(M, prefs):
    for p in prefs:
        if M % p == 0:
            return p
    raise ValueError(f"no block size for {M}")


def _mm_pool_body(r_ref, s_ref, o_ref, rs_ref, acc_ref, *, ksteps, bm, bn):
    mi = pl.program_id(0)
    ni = pl.program_id(1)
    ki = pl.program_id(2)

    @pl.when(ki == 0)
    def _():
        acc_ref[...] = jnp.zeros_like(acc_ref)

    acc_ref[...] += jnp.dot(r_ref[...], s_ref[...],
                            preferred_element_type=jnp.float32)

    @pl.when(ki == ksteps - 1)
    def _():
        acc = acc_ref[...]
        rows = mi * bm + jax.lax.broadcasted_iota(jnp.int32, (bm, bn), 0)
        cols = ni * bn + jax.lax.broadcasted_iota(jnp.int32, (bm, bn), 1)
        acc = jnp.where(rows == cols, 0.0, acc)
        o_ref[...] = acc.astype(jnp.bfloat16)

        @pl.when(ni == 0)
        def _():
            rs_ref[...] = jnp.zeros_like(rs_ref)

        rs_ref[...] += jnp.sum(acc, axis=1, keepdims=True)


def _mm_pool(R, S):
    M, K = R.shape
    K2_, N = S.shape
    bm = _pick(M, (512, 256, 128))
    bn = _pick(N, (512, 256, 128))
    bk = _pick(K, (512, 256, 128))
    assert K == K2_
    grid = (M // bm, N // bn, K // bk)
    return pl.pallas_call(
        functools.partial(_mm_pool_body, ksteps=K // bk, bm=bm, bn=bn),
        grid=grid,
        in_specs=[
            pl.BlockSpec((bm, bk), lambda m, n, k: (m, k)),
            pl.BlockSpec((bk, bn), lambda m, n, k: (k, n)),
        ],
        out_specs=[
            pl.BlockSpec((bm, bn), lambda m, n, k: (m, n)),
            pl.BlockSpec((bm, 1), lambda m, n, k: (m, 0)),
        ],
        out_shape=[
            jax.ShapeDtypeStruct((M, N), jnp.bfloat16),
            jax.ShapeDtypeStruct((M, 1), jnp.float32),
        ],
        scratch_shapes=[pltpu.VMEM((bm, bn), jnp.float32)],
        compiler_params=pltpu.CompilerParams(
            dimension_semantics=("parallel", "parallel", "arbitrary")),
    )(R, S)


def _mm_onehot_body(a_ref, pm_ref, o_ref, acc_ref, *, ksteps, bm, bk, bn):
    mi = pl.program_id(0)
    ki = pl.program_id(2)

    @pl.when(ki == 0)
    def _():
        acc_ref[...] = jnp.zeros_like(acc_ref)

    rows = ki * bk + jax.lax.broadcasted_iota(jnp.int32, (bk, bn), 0)
    p = jnp.where(rows == pm_ref[...], 1.0, 0.0).astype(jnp.bfloat16)
    acc_ref[...] += jnp.dot(a_ref[...], p, preferred_element_type=jnp.float32)

    @pl.when(ki == ksteps - 1)
    def _():
        mrows = mi * bm + jax.lax.broadcasted_iota(jnp.int32, (bm, bn), 0)
        eye = jnp.where(mrows == pm_ref[...], 1.0, 0.0)
        o_ref[...] = (acc_ref[...] + eye).astype(jnp.bfloat16)


def _mm_onehot(A, perm_p):
    M = A.shape[0]
    N = perm_p.shape[1]
    bm = _pick(M, (512, 256, 128))
    bn = _pick(N, (512, 256, 128))
    bk = _pick(M, (512, 256, 128))
    grid = (M // bm, N // bn, M // bk)
    return pl.pallas_call(
        functools.partial(_mm_onehot_body, ksteps=M // bk, bm=bm, bk=bk, bn=bn),
        grid=grid,
        in_specs=[
            pl.BlockSpec((bm, bk), lambda m, n, k: (m, k)),
            pl.BlockSpec((1, bn), lambda m, n, k: (0, n)),
        ],
        out_specs=pl.BlockSpec((bm, bn), lambda m, n, k: (m, n)),
        out_shape=jax.ShapeDtypeStruct((M, N), jnp.bfloat16),
        scratch_shapes=[pltpu.VMEM((bm, bn), jnp.float32)],
        compiler_params=pltpu.CompilerParams(
            dimension_semantics=("parallel", "parallel", "arbitrary")),
    )(A, perm_p)


def _mm_agg_body(a_ref, u_ref, um_ref, dinv_ref, b_ref, o_ref, acc_ref,
                 *, ksteps, relu):
    ki = pl.program_id(1)

    @pl.when(ki == 0)
    def _():
        acc_ref[...] = jnp.zeros_like(acc_ref)

    u = u_ref[...]
    uh = u.astype(jnp.bfloat16)
    ul = (u - uh.astype(jnp.float32)).astype(jnp.bfloat16)
    a = a_ref[...]
    acc_ref[...] += (jnp.dot(a, uh, preferred_element_type=jnp.float32)
                     + jnp.dot(a, ul, preferred_element_type=jnp.float32))

    @pl.when(ki == ksteps - 1)
    def _():
        out = dinv_ref[...] * (acc_ref[...] + 2.0 * um_ref[...]) + b_ref[...]
        if relu:
            out = jnp.maximum(out, 0.0)
        o_ref[...] = out


def _mm_agg(A, u, dinv, b, relu):
    M, C = u.shape
    bm = _pick(M, (512, 256, 128))
    bk = _pick(M, (512, 256, 128))
    assert A.shape == (M, M)
    grid = (M // bm, M // bk)
    return pl.pallas_call(
        functools.partial(_mm_agg_body, ksteps=M // bk, relu=relu),
        grid=grid,
        in_specs=[
            pl.BlockSpec((bm, bk), lambda m, k: (m, k)),
            pl.BlockSpec((bk, C), lambda m, k: (k, 0)),
            pl.BlockSpec((bm, C), lambda m, k: (m, 0)),
            pl.BlockSpec((bm, 1), lambda m, k: (m, 0)),
            pl.BlockSpec((1, C), lambda m, k: (0, 0)),
        ],
        out_specs=pl.BlockSpec((bm, C), lambda m, k: (m, 0)),
        out_shape=jax.ShapeDtypeStruct((M, C), jnp.float32),
        scratch_shapes=[pltpu.VMEM((bm, C), jnp.float32)],
        compiler_params=pltpu.CompilerParams(
            dimension_semantics=("parallel", "arbitrary")),
    )(A, u, u, dinv, b)


def _mm_xw_body(x_ref, w_ref, dinv_ref, o_ref):
    o_ref[...] = dinv_ref[...] * jnp.dot(x_ref[...], w_ref[...],
                                         preferred_element_type=jnp.float32)


def _mm_xw(x, W, dinv):
    M, K = x.shape
    C = W.shape[1]
    bm = _pick(M, (1024, 512, 256, 128))
    return pl.pallas_call(
        _mm_xw_body,
        grid=(M // bm,),
        in_specs=[
            pl.BlockSpec((bm, K), lambda m: (m, 0)),
            pl.BlockSpec((K, C), lambda m: (0, 0)),
            pl.BlockSpec((bm, 1), lambda m: (m, 0)),
        ],
        out_specs=pl.BlockSpec((bm, C), lambda m: (m, 0)),
        out_shape=jax.ShapeDtypeStruct((M, C), jnp.float32),
        compiler_params=pltpu.CompilerParams(
            dimension_semantics=("parallel",)),
    )(x, W, dinv)


def _pad_rows(a, rows):
    return jnp.pad(a, ((0, rows - a.shape[0]), (0, 0)))


def _score_topk(h, p, n_valid, k):
    s = jnp.tanh((h[:n_valid] @ p) / jnp.linalg.norm(p))
    _, perm = jax.lax.top_k(s, k)
    return s, perm


def _pool_level(A_p, h_p, s, perm, np_, kp):
    k = perm.shape[0]
    R = A_p[perm]
    R = R.at[jnp.arange(k), perm].set(jnp.bfloat16(1.0))
    R = _pad_rows(R, kp)
    perm_p = jnp.full((1, kp), -1, jnp.int32).at[0, :k].set(perm)
    S = _mm_onehot(A_p, perm_p)
    A_next, rs = _mm_pool(R, S)
    dinv = (rs + 2.0) ** -0.5
    hp = _pad_rows(h_p[perm] * s[perm][:, None], kp)
    return A_next, dinv, hp


def kernel(x, edge_index, W0, b0, W1, b1, W2, b2, W3, b3,
           p1, p2, p3, U0, ub0, U1, ub1, U2, ub2):
    n = N_NODES
    ei = edge_index.astype(jnp.int32)
    dst, src = ei[1], ei[0]

    deg0 = jnp.zeros((n,), jnp.float32).at[dst].add(1.0) + 2.0
    dinv0 = deg0 ** -0.5
    dinv0_p = _pad_rows(dinv0[:, None], NP)
    x_p = _pad_rows(x, NP)

    def gcn0(h_p, W, b, relu):
        u_p = _mm_xw(h_p, W, dinv0_p)
        agg = jnp.zeros_like(u_p).at[dst].add(u_p[src])
        out = dinv0_p * (agg + 2.0 * u_p) + b[None, :]
        return jnp.maximum(out, 0.0) if relu else out

    h0_p = gcn0(x_p, W0, b0, True)

    s1, perm1 = _score_topk(h0_p, p1, n, K1)
    inv1 = jnp.full((n,), -1, jnp.int32).at[perm1].set(
        jnp.arange(K1, dtype=jnp.int32))
    nd = dst != src
    r_rows = jnp.where(nd, inv1[dst], -1)
    R = jnp.zeros((K1P, NP), jnp.float32).at[r_rows, src].add(
        1.0, mode="drop")
    R = R.at[jnp.arange(K1), perm1].add(1.0)
    s_cols = jnp.where(nd, inv1[src], -1)
    S = jnp.zeros((NP, K1P), jnp.float32).at[dst, s_cols].add(
        1.0, mode="drop")
    S = S.at[perm1, jnp.arange(K1)].add(1.0)
    A1, rs1 = _mm_pool(R.astype(jnp.bfloat16), S.astype(jnp.bfloat16))
    dinv1 = (rs1 + 2.0) ** -0.5
    h1p = _pad_rows(h0_p[perm1] * s1[perm1][:, None], K1P)

    def gcn(h_p, A_p, dinv_p, W, b, relu):
        u_p = _mm_xw(h_p, W, dinv_p)
        return _mm_agg(A_p, u_p, dinv_p, b[None, :], relu)

    h1 = gcn(h1p, A1, dinv1, W1, b1, True)

    s2, perm2 = _score_topk(h1, p2, K1, K2)
    A2, dinv2, h2p = _pool_level(A1, h1, s2, perm2, K1P, K2P)
    h2 = gcn(h2p, A2, dinv2, W2, b2, True)

    s3, perm3 = _score_topk(h2, p3, K2, K3)
    A3, dinv3, h3p = _pool_level(A2, h2, s3, perm3, K2P, K3P)
    h = gcn(h3p, A3, dinv3, W3, b3, True)

    h = h2.at[perm3].add(h[:K3])
    h = gcn(h, A2, dinv2, U0, ub0, True)
    h = h1.at[perm2].add(h[:K2])
    h = gcn(h, A1, dinv1, U1, ub1, True)
    h = h0_p.at[perm1].add(h[:K1])
    h = gcn0(h, U2, ub2, False)
    return h[:n]


# speedup vs baseline: 1.3687x; 1.3687x over previous
import functools
import math

import jax
import jax.numpy as jnp
import numpy as np
from jax.experimental import pallas as pl
from jax.experimental.pallas import tpu as pltpu

N_NODES = 10000
RATIO = 0.5

NP = 10240
K1, K2, K3 = 5000, 2500, 1250
K1P, K2P, K3P = 5120, 2560, 1280


def _pick(M, prefs):
    for p in prefs:
        if M % p == 0:
            return p
    raise ValueError(f"no block size for {M}")


def _mm_pool_body(g_ref, h_ref, ap_ref, dr_ref, dc_ref, o_ref, rs_ref,
                  acc_ref, *, ksteps, bm, bn):
    mi = pl.program_id(0)
    ni = pl.program_id(1)
    ki = pl.program_id(2)

    @pl.when(ki == 0)
    def _():
        acc_ref[...] = jnp.zeros_like(acc_ref)

    acc_ref[...] += jnp.dot(g_ref[...], h_ref[...],
                            preferred_element_type=jnp.float32)

    @pl.when(ki == ksteps - 1)
    def _():
        corr = ap_ref[...].astype(jnp.float32) * (2.0 - dr_ref[...] - dc_ref[...])
        acc = acc_ref[...] + corr
        rows = mi * bm + jax.lax.broadcasted_iota(jnp.int32, (bm, bn), 0)
        cols = ni * bn + jax.lax.broadcasted_iota(jnp.int32, (bm, bn), 1)
        acc = jnp.where(rows == cols, 0.0, acc)
        o_ref[...] = acc.astype(jnp.bfloat16)

        @pl.when(ni == 0)
        def _():
            rs_ref[...] = jnp.zeros_like(rs_ref)

        rs_ref[...] += jnp.sum(acc, axis=1, keepdims=True)


def _mm_pool(G, H, Apool, dr, dc):
    M, K = G.shape
    _, N = H.shape
    bm = _pick(M, (512, 256, 128))
    bn = _pick(N, (512, 256, 128))
    bk = _pick(K, (512, 256, 128))
    grid = (M // bm, N // bn, K // bk)
    return pl.pallas_call(
        functools.partial(_mm_pool_body, ksteps=K // bk, bm=bm, bn=bn),
        grid=grid,
        in_specs=[
            pl.BlockSpec((bm, bk), lambda m, n, k: (m, k)),
            pl.BlockSpec((bk, bn), lambda m, n, k: (k, n)),
            pl.BlockSpec((bm, bn), lambda m, n, k: (m, n)),
            pl.BlockSpec((bm, 1), lambda m, n, k: (m, 0)),
            pl.BlockSpec((1, bn), lambda m, n, k: (0, n)),
        ],
        out_specs=[
            pl.BlockSpec((bm, bn), lambda m, n, k: (m, n)),
            pl.BlockSpec((bm, 1), lambda m, n, k: (m, 0)),
        ],
        out_shape=[
            jax.ShapeDtypeStruct((M, N), jnp.bfloat16),
            jax.ShapeDtypeStruct((M, 1), jnp.float32),
        ],
        scratch_shapes=[pltpu.VMEM((bm, bn), jnp.float32)],
        compiler_params=pltpu.CompilerParams(
            dimension_semantics=("parallel", "parallel", "arbitrary")),
    )(G, H, Apool, dr, dc)


def _mm_body(a_ref, b_ref, o_ref, acc_ref, *, ksteps):
    ki = pl.program_id(2)

    @pl.when(ki == 0)
    def _():
        acc_ref[...] = jnp.zeros_like(acc_ref)

    acc_ref[...] += jnp.dot(a_ref[...], b_ref[...],
                            preferred_element_type=jnp.float32)

    @pl.when(ki == ksteps - 1)
    def _():
        o_ref[...] = acc_ref[...].astype(jnp.bfloat16)


def _mm(A, B):
    M, K = A.shape
    _, N = B.shape
    bm = _pick(M, (512, 256, 128))
    bn = _pick(N, (512, 256, 128))
    bk = _pick(K, (512, 256, 128))
    grid = (M // bm, N // bn, K // bk)
    return pl.pallas_call(
        functools.partial(_mm_body, ksteps=K // bk),
        grid=grid,
        in_specs=[
            pl.BlockSpec((bm, bk), lambda m, n, k: (m, k)),
            pl.BlockSpec((bk, bn), lambda m, n, k: (k, n)),
        ],
        out_specs=pl.BlockSpec((bm, bn), lambda m, n, k: (m, n)),
        out_shape=jax.ShapeDtypeStruct((M, N), jnp.bfloat16),
        scratch_shapes=[pltpu.VMEM((bm, bn), jnp.float32)],
        compiler_params=pltpu.CompilerParams(
            dimension_semantics=("parallel", "parallel", "arbitrary")),
    )(A, B)


def _onehot_body(pm_ref, o_ref, *, bm, bn):
    mi = pl.program_id(0)
    rows = mi * bm + jax.lax.broadcasted_iota(jnp.int32, (bm, bn), 0)
    o_ref[...] = jnp.where(rows == pm_ref[...], 1.0, 0.0).astype(jnp.bfloat16)


def _onehot(perm_p, M):
    N = perm_p.shape[1]
    bm = _pick(M, (1024, 512, 256))
    bn = _pick(N, (512, 256, 128))
    return pl.pallas_call(
        functools.partial(_onehot_body, bm=bm, bn=bn),
        grid=(M // bm, N // bn),
        in_specs=[pl.BlockSpec((1, bn), lambda m, n: (0, n))],
        out_specs=pl.BlockSpec((bm, bn), lambda m, n: (m, n)),
        out_shape=jax.ShapeDtypeStruct((M, N), jnp.bfloat16),
        compiler_params=pltpu.CompilerParams(
            dimension_semantics=("parallel", "parallel")),
    )(perm_p)


def _mm_agg_body(a_ref, u_ref, um_ref, dinv_ref, b_ref, o_ref, acc_ref,
                 *, ksteps, relu):
    ki = pl.program_id(1)

    @pl.when(ki == 0)
    def _():
        acc_ref[...] = jnp.zeros_like(acc_ref)

    u = u_ref[...]
    uh = u.astype(jnp.bfloat16)
    ul = (u - uh.astype(jnp.float32)).astype(jnp.bfloat16)
    a = a_ref[...]
    acc_ref[...] += (jnp.dot(a, uh, preferred_element_type=jnp.float32)
                     + jnp.dot(a, ul, preferred_element_type=jnp.float32))

    @pl.when(ki == ksteps - 1)
    def _():
        out = dinv_ref[...] * (acc_ref[...] + 2.0 * um_ref[...]) + b_ref[...]
        if relu:
            out = jnp.maximum(out, 0.0)
        o_ref[...] = out


def _mm_agg(A, u, dinv, b, relu):
    M, C = u.shape
    bm = _pick(M, (512, 256, 128))
    bk = _pick(M, (512, 256, 128))
    assert A.shape == (M, M)
    grid = (M // bm, M // bk)
    return pl.pallas_call(
        functools.partial(_mm_agg_body, ksteps=M // bk, relu=relu),
        grid=grid,
        in_specs=[
            pl.BlockSpec((bm, bk), lambda m, k: (m, k)),
            pl.BlockSpec((bk, C), lambda m, k: (k, 0)),
            pl.BlockSpec((bm, C), lambda m, k: (m, 0)),
            pl.BlockSpec((bm, 1), lambda m, k: (m, 0)),
            pl.BlockSpec((1, C), lambda m, k: (0, 0)),
        ],
        out_specs=pl.BlockSpec((bm, C), lambda m, k: (m, 0)),
        out_shape=jax.ShapeDtypeStruct((M, C), jnp.float32),
        scratch_shapes=[pltpu.VMEM((bm, C), jnp.float32)],
        compiler_params=pltpu.CompilerParams(
            dimension_semantics=("parallel", "arbitrary")),
    )(A, u, u, dinv, b)


def _mm_xw_body(x_ref, w_ref, dinv_ref, o_ref):
    o_ref[...] = dinv_ref[...] * jnp.dot(x_ref[...], w_ref[...],
                                         preferred_element_type=jnp.float32)


def _mm_xw(x, W, dinv):
    M, K = x.shape
    C = W.shape[1]
    bm = _pick(M, (1024, 512, 256, 128))
    return pl.pallas_call(
        _mm_xw_body,
        grid=(M // bm,),
        in_specs=[
            pl.BlockSpec((bm, K), lambda m: (m, 0)),
            pl.BlockSpec((K, C), lambda m: (0, 0)),
            pl.BlockSpec((bm, 1), lambda m: (m, 0)),
        ],
        out_specs=pl.BlockSpec((bm, C), lambda m: (m, 0)),
        out_shape=jax.ShapeDtypeStruct((M, C), jnp.float32),
        compiler_params=pltpu.CompilerParams(
            dimension_semantics=("parallel",)),
    )(x, W, dinv)


def _pad_rows(a, rows):
    return jnp.pad(a, ((0, rows - a.shape[0]), (0, 0)))


def _score_topk(h, p, n_valid, k):
    s = jnp.tanh((h[:n_valid] @ p) / jnp.linalg.norm(p))
    _, perm = jax.lax.top_k(s, k)
    return s, perm


def _pool_level(A_p, perm, kp, dself=None):
    k = perm.shape[0]
    G = _pad_rows(A_p[perm], kp)
    perm_p = jnp.full((1, kp), -1, jnp.int32).at[0, :k].set(perm)
    H = _mm(A_p, _onehot(perm_p, A_p.shape[0]))
    Apool = _pad_rows(H[perm], kp)
    if dself is None:
        dr = jnp.zeros((kp, 1), jnp.float32)
        dc = jnp.zeros((1, kp), jnp.float32)
    else:
        dp = jnp.pad(dself[perm], (0, kp - k))
        dr = dp[:, None]
        dc = dp[None, :]
    C, rs = _mm_pool(G, H, Apool, dr, dc)
    dinv = (rs + 2.0) ** -0.5
    return C, dinv


def kernel(x, edge_index, W0, b0, W1, b1, W2, b2, W3, b3,
           p1, p2, p3, U0, ub0, U1, ub1, U2, ub2):
    n = N_NODES
    ei = edge_index.astype(jnp.int32)
    dst, src = ei[1], ei[0]

    A = jnp.zeros((NP, NP), jnp.float32).at[dst, src].add(1.0)
    Ab = A.astype(jnp.bfloat16)
    deg0 = jnp.zeros((n,), jnp.float32).at[dst].add(1.0) + 2.0
    dself = jnp.zeros((n,), jnp.float32).at[dst].add(
        (dst == src).astype(jnp.float32))
    dinv0_p = _pad_rows((deg0 ** -0.5)[:, None], NP)
    x_p = _pad_rows(x, NP)

    def gcn(h_p, A_p, dinv_p, W, b, relu):
        u_p = _mm_xw(h_p, W, dinv_p)
        return _mm_agg(A_p, u_p, dinv_p, b[None, :], relu)

    h0_p = gcn(x_p, Ab, dinv0_p, W0, b0, True)

    s1, perm1 = _score_topk(h0_p, p1, n, K1)
    A1, dinv1 = _pool_level(Ab, perm1, K1P, dself=dself)
    h1p = _pad_rows(h0_p[perm1] * s1[perm1][:, None], K1P)
    h1 = gcn(h1p, A1, dinv1, W1, b1, True)

    s2, perm2 = _score_topk(h1, p2, K1, K2)
    A2, dinv2 = _pool_level(A1, perm2, K2P)
    h2p = _pad_rows(h1[perm2] * s2[perm2][:, None], K2P)
    h2 = gcn(h2p, A2, dinv2, W2, b2, True)

    s3, perm3 = _score_topk(h2, p3, K2, K3)
    A3, dinv3 = _pool_level(A2, perm3, K3P)
    h3p = _pad_rows(h2[perm3] * s3[perm3][:, None], K3P)
    h = gcn(h3p, A3, dinv3, W3, b3, True)

    h = h2.at[perm3].add(h[:K3])
    h = gcn(h, A2, dinv2, U0, ub0, True)
    h = h1.at[perm2].add(h[:K2])
    h = gcn(h, A1, dinv1, U1, ub1, True)
    h = h0_p.at[perm1].add(h[:K1])
    h = gcn(h, Ab, dinv0_p, U2, ub2, False)
    return h[:n]


# speedup vs baseline: 2.4137x; 1.7634x over previous
import functools
import math

import jax
import jax.numpy as jnp
import numpy as np
from jax.experimental import pallas as pl
from jax.experimental.pallas import tpu as pltpu

N_NODES = 10000
RATIO = 0.5

NP = 10240
K1, K2, K3 = 5000, 2500, 1250
K1P, K2P, K3P = 5120, 2560, 1280


def _pick(M, prefs):
    for p in prefs:
        if M % p == 0:
            return p
    raise ValueError(f"no block size for {M}")


def _mm_pool_body(g_ref, h_ref, ap_ref, dr_ref, dc_ref, o_ref, rs_ref,
                  acc_ref, *, ksteps, bm, bn):
    mi = pl.program_id(0)
    ni = pl.program_id(1)
    ki = pl.program_id(2)

    @pl.when(ki == 0)
    def _():
        acc_ref[...] = jnp.zeros_like(acc_ref)

    acc_ref[...] += jnp.dot(g_ref[...], h_ref[...],
                            preferred_element_type=jnp.float32)

    @pl.when(ki == ksteps - 1)
    def _():
        corr = ap_ref[...].astype(jnp.float32) * (2.0 - dr_ref[...] - dc_ref[...])
        acc = acc_ref[...] + corr
        rows = mi * bm + jax.lax.broadcasted_iota(jnp.int32, (bm, bn), 0)
        cols = ni * bn + jax.lax.broadcasted_iota(jnp.int32, (bm, bn), 1)
        acc = jnp.where(rows == cols, 0.0, acc)
        o_ref[...] = acc.astype(jnp.bfloat16)

        @pl.when(ni == 0)
        def _():
            rs_ref[...] = jnp.zeros_like(rs_ref)

        rs_ref[...] += jnp.sum(acc, axis=1, keepdims=True)


def _mm_pool(G, H, Apool, dr, dc):
    M, K = G.shape
    _, N = H.shape
    bm = _pick(M, (1024, 512, 256, 128))
    bn = _pick(N, (1024, 512, 256, 128))
    bk = _pick(K, (1024, 512, 256, 128))
    grid = (M // bm, N // bn, K // bk)
    return pl.pallas_call(
        functools.partial(_mm_pool_body, ksteps=K // bk, bm=bm, bn=bn),
        grid=grid,
        in_specs=[
            pl.BlockSpec((bm, bk), lambda m, n, k: (m, k)),
            pl.BlockSpec((bk, bn), lambda m, n, k: (k, n)),
            pl.BlockSpec((bm, bn), lambda m, n, k: (m, n)),
            pl.BlockSpec((bm, 1), lambda m, n, k: (m, 0)),
            pl.BlockSpec((1, bn), lambda m, n, k: (0, n)),
        ],
        out_specs=[
            pl.BlockSpec((bm, bn), lambda m, n, k: (m, n)),
            pl.BlockSpec((bm, 1), lambda m, n, k: (m, 0)),
        ],
        out_shape=[
            jax.ShapeDtypeStruct((M, N), jnp.bfloat16),
            jax.ShapeDtypeStruct((M, 1), jnp.float32),
        ],
        scratch_shapes=[pltpu.VMEM((bm, bn), jnp.float32)],
        compiler_params=pltpu.CompilerParams(
            dimension_semantics=("parallel", "parallel", "arbitrary")),
    )(G, H, Apool, dr, dc)


def _mm_body(a_ref, b_ref, o_ref, acc_ref, *, ksteps):
    ki = pl.program_id(2)

    @pl.when(ki == 0)
    def _():
        acc_ref[...] = jnp.zeros_like(acc_ref)

    acc_ref[...] += jnp.dot(a_ref[...], b_ref[...],
                            preferred_element_type=jnp.float32)

    @pl.when(ki == ksteps - 1)
    def _():
        o_ref[...] = acc_ref[...].astype(jnp.bfloat16)


def _mm(A, B):
    M, K = A.shape
    _, N = B.shape
    bm = _pick(M, (1024, 512, 256, 128))
    bn = _pick(N, (1024, 512, 256, 128))
    bk = _pick(K, (1024, 512, 256, 128))
    grid = (M // bm, N // bn, K // bk)
    return pl.pallas_call(
        functools.partial(_mm_body, ksteps=K // bk),
        grid=grid,
        in_specs=[
            pl.BlockSpec((bm, bk), lambda m, n, k: (m, k)),
            pl.BlockSpec((bk, bn), lambda m, n, k: (k, n)),
        ],
        out_specs=pl.BlockSpec((bm, bn), lambda m, n, k: (m, n)),
        out_shape=jax.ShapeDtypeStruct((M, N), jnp.bfloat16),
        scratch_shapes=[pltpu.VMEM((bm, bn), jnp.float32)],
        compiler_params=pltpu.CompilerParams(
            dimension_semantics=("parallel", "parallel", "arbitrary")),
    )(A, B)


def _onehot_body(pm_ref, o_ref, *, bm, bn):
    mi = pl.program_id(0)
    rows = mi * bm + jax.lax.broadcasted_iota(jnp.int32, (bm, bn), 0)
    o_ref[...] = jnp.where(rows == pm_ref[...], 1.0, 0.0).astype(jnp.bfloat16)


def _onehot(perm_p, M):
    N = perm_p.shape[1]
    bm = _pick(M, (1024, 512, 256))
    bn = _pick(N, (512, 256, 128))
    return pl.pallas_call(
        functools.partial(_onehot_body, bm=bm, bn=bn),
        grid=(M // bm, N // bn),
        in_specs=[pl.BlockSpec((1, bn), lambda m, n: (0, n))],
        out_specs=pl.BlockSpec((bm, bn), lambda m, n: (m, n)),
        out_shape=jax.ShapeDtypeStruct((M, N), jnp.bfloat16),
        compiler_params=pltpu.CompilerParams(
            dimension_semantics=("parallel", "parallel")),
    )(perm_p)


def _mm_agg_body(a_ref, u_ref, um_ref, dinv_ref, b_ref, o_ref, acc_ref,
                 *, ksteps, relu):
    ki = pl.program_id(1)

    @pl.when(ki == 0)
    def _():
        acc_ref[...] = jnp.zeros_like(acc_ref)

    u = u_ref[...]
    uh = u.astype(jnp.bfloat16)
    ul = (u - uh.astype(jnp.float32)).astype(jnp.bfloat16)
    a = a_ref[...]
    acc_ref[...] += (jnp.dot(a, uh, preferred_element_type=jnp.float32)
                     + jnp.dot(a, ul, preferred_element_type=jnp.float32))

    @pl.when(ki == ksteps - 1)
    def _():
        out = dinv_ref[...] * (acc_ref[...] + 2.0 * um_ref[...]) + b_ref[...]
        if relu:
            out = jnp.maximum(out, 0.0)
        o_ref[...] = out


def _mm_agg(A, u, dinv, b, relu):
    M, C = u.shape
    bm = _pick(M, (1024, 512, 256, 128))
    bk = _pick(M, (1024, 512, 256, 128))
    assert A.shape == (M, M)
    grid = (M // bm, M // bk)
    return pl.pallas_call(
        functools.partial(_mm_agg_body, ksteps=M // bk, relu=relu),
        grid=grid,
        in_specs=[
            pl.BlockSpec((bm, bk), lambda m, k: (m, k)),
            pl.BlockSpec((bk, C), lambda m, k: (k, 0)),
            pl.BlockSpec((bm, C), lambda m, k: (m, 0)),
            pl.BlockSpec((bm, 1), lambda m, k: (m, 0)),
            pl.BlockSpec((1, C), lambda m, k: (0, 0)),
        ],
        out_specs=pl.BlockSpec((bm, C), lambda m, k: (m, 0)),
        out_shape=jax.ShapeDtypeStruct((M, C), jnp.float32),
        scratch_shapes=[pltpu.VMEM((bm, C), jnp.float32)],
        compiler_params=pltpu.CompilerParams(
            dimension_semantics=("parallel", "arbitrary")),
    )(A, u, u, dinv, b)


def _mm_xw_body(x_ref, w_ref, dinv_ref, o_ref):
    o_ref[...] = dinv_ref[...] * jnp.dot(x_ref[...], w_ref[...],
                                         preferred_element_type=jnp.float32)


def _mm_xw(x, W, dinv):
    M, K = x.shape
    C = W.shape[1]
    bm = _pick(M, (1024, 512, 256, 128))
    return pl.pallas_call(
        _mm_xw_body,
        grid=(M // bm,),
        in_specs=[
            pl.BlockSpec((bm, K), lambda m: (m, 0)),
            pl.BlockSpec((K, C), lambda m: (0, 0)),
            pl.BlockSpec((bm, 1), lambda m: (m, 0)),
        ],
        out_specs=pl.BlockSpec((bm, C), lambda m: (m, 0)),
        out_shape=jax.ShapeDtypeStruct((M, C), jnp.float32),
        compiler_params=pltpu.CompilerParams(
            dimension_semantics=("parallel",)),
    )(x, W, dinv)


def _pad_rows(a, rows):
    return jnp.pad(a, ((0, rows - a.shape[0]), (0, 0)))


def _score_topk(h, p, n_valid, k):
    s = jnp.tanh((h[:n_valid] @ p) / jnp.linalg.norm(p))
    _, perm = jax.lax.top_k(s, k)
    return s, perm


def _pool_level(A_p, perm, kp, dself=None):
    k = perm.shape[0]
    G = _pad_rows(A_p[perm], kp)
    perm_p = jnp.full((1, kp), -1, jnp.int32).at[0, :k].set(perm)
    H = _mm(A_p, _onehot(perm_p, A_p.shape[0]))
    Apool = _pad_rows(H[perm], kp)
    if dself is None:
        dr = jnp.zeros((kp, 1), jnp.float32)
        dc = jnp.zeros((1, kp), jnp.float32)
    else:
        dp = jnp.pad(dself[perm], (0, kp - k))
        dr = dp[:, None]
        dc = dp[None, :]
    C, rs = _mm_pool(G, H, Apool, dr, dc)
    dinv = (rs + 2.0) ** -0.5
    return C, dinv


def kernel(x, edge_index, W0, b0, W1, b1, W2, b2, W3, b3,
           p1, p2, p3, U0, ub0, U1, ub1, U2, ub2):
    n = N_NODES
    ei = edge_index.astype(jnp.int32)
    dst, src = ei[1], ei[0]

    A = jnp.zeros((NP, NP), jnp.float32).at[dst, src].add(1.0)
    Ab = A.astype(jnp.bfloat16)
    deg0 = jnp.zeros((n,), jnp.float32).at[dst].add(1.0) + 2.0
    dself = jnp.zeros((n,), jnp.float32).at[dst].add(
        (dst == src).astype(jnp.float32))
    dinv0_p = _pad_rows((deg0 ** -0.5)[:, None], NP)
    x_p = _pad_rows(x, NP)

    def gcn(h_p, A_p, dinv_p, W, b, relu):
        u_p = _mm_xw(h_p, W, dinv_p)
        return _mm_agg(A_p, u_p, dinv_p, b[None, :], relu)

    h0_p = gcn(x_p, Ab, dinv0_p, W0, b0, True)

    s1, perm1 = _score_topk(h0_p, p1, n, K1)
    A1, dinv1 = _pool_level(Ab, perm1, K1P, dself=dself)
    h1p = _pad_rows(h0_p[perm1] * s1[perm1][:, None], K1P)
    h1 = gcn(h1p, A1, dinv1, W1, b1, True)

    s2, perm2 = _score_topk(h1, p2, K1, K2)
    A2, dinv2 = _pool_level(A1, perm2, K2P)
    h2p = _pad_rows(h1[perm2] * s2[perm2][:, None], K2P)
    h2 = gcn(h2p, A2, dinv2, W2, b2, True)

    s3, perm3 = _score_topk(h2, p3, K2, K3)
    A3, dinv3 = _pool_level(A2, perm3, K3P)
    h3p = _pad_rows(h2[perm3] * s3[perm3][:, None], K3P)
    h = gcn(h3p, A3, dinv3, W3, b3, True)

    h = h2.at[perm3].add(h[:K3])
    h = gcn(h, A2, dinv2, U0, ub0, True)
    h = h1.at[perm2].add(h[:K2])
    h = gcn(h, A1, dinv1, U1, ub1, True)
    h = h0_p.at[perm1].add(h[:K1])
    h = gcn(h, Ab, dinv0_p, U2, ub2, False)
    return h[:n]


# speedup vs baseline: 2.5406x; 1.0526x over previous
import functools
import math

import jax
import jax.numpy as jnp
import numpy as np
from jax.experimental import pallas as pl
from jax.experimental.pallas import tpu as pltpu

N_NODES = 10000
RATIO = 0.5

NP = 10240
K1, K2, K3 = 5000, 2500, 1250
K1P, K2P, K3P = 5120, 2560, 1280


def _pick(M, prefs):
    for p in prefs:
        if M % p == 0:
            return p
    raise ValueError(f"no block size for {M}")


def _mm_pool_body(g_ref, h_ref, ap_ref, dr_ref, dc_ref, o_ref, rs_ref,
                  acc_ref, *, ksteps, bm, bn):
    mi = pl.program_id(0)
    ni = pl.program_id(1)
    ki = pl.program_id(2)

    @pl.when(ki == 0)
    def _():
        acc_ref[...] = jnp.zeros_like(acc_ref)

    acc_ref[...] += jnp.dot(g_ref[...], h_ref[...],
                            preferred_element_type=jnp.float32)

    @pl.when(ki == ksteps - 1)
    def _():
        corr = ap_ref[...].astype(jnp.float32) * (2.0 - dr_ref[...] - dc_ref[...])
        acc = acc_ref[...] + corr
        rows = mi * bm + jax.lax.broadcasted_iota(jnp.int32, (bm, bn), 0)
        cols = ni * bn + jax.lax.broadcasted_iota(jnp.int32, (bm, bn), 1)
        acc = jnp.where(rows == cols, 0.0, acc)
        o_ref[...] = acc.astype(jnp.bfloat16)

        @pl.when(ni == 0)
        def _():
            rs_ref[...] = jnp.zeros_like(rs_ref)

        rs_ref[...] += jnp.sum(acc, axis=1, keepdims=True)


def _mm_pool(G, H, Apool, dr, dc):
    M, K = G.shape
    _, N = H.shape
    bm = _pick(M, (1024, 512, 256, 128))
    bn = _pick(N, (1024, 512, 256, 128))
    bk = _pick(K, (1024, 512, 256, 128))
    grid = (M // bm, N // bn, K // bk)
    return pl.pallas_call(
        functools.partial(_mm_pool_body, ksteps=K // bk, bm=bm, bn=bn),
        grid=grid,
        in_specs=[
            pl.BlockSpec((bm, bk), lambda m, n, k: (m, k)),
            pl.BlockSpec((bk, bn), lambda m, n, k: (k, n)),
            pl.BlockSpec((bm, bn), lambda m, n, k: (m, n)),
            pl.BlockSpec((bm, 1), lambda m, n, k: (m, 0)),
            pl.BlockSpec((1, bn), lambda m, n, k: (0, n)),
        ],
        out_specs=[
            pl.BlockSpec((bm, bn), lambda m, n, k: (m, n)),
            pl.BlockSpec((bm, 1), lambda m, n, k: (m, 0)),
        ],
        out_shape=[
            jax.ShapeDtypeStruct((M, N), jnp.bfloat16),
            jax.ShapeDtypeStruct((M, 1), jnp.float32),
        ],
        scratch_shapes=[pltpu.VMEM((bm, bn), jnp.float32)],
        compiler_params=pltpu.CompilerParams(
            dimension_semantics=("parallel", "parallel", "arbitrary")),
    )(G, H, Apool, dr, dc)


def _mm_body(a_ref, b_ref, o_ref, acc_ref, *, ksteps):
    ki = pl.program_id(2)

    @pl.when(ki == 0)
    def _():
        acc_ref[...] = jnp.zeros_like(acc_ref)

    acc_ref[...] += jnp.dot(a_ref[...], b_ref[...],
                            preferred_element_type=jnp.float32)

    @pl.when(ki == ksteps - 1)
    def _():
        o_ref[...] = acc_ref[...].astype(jnp.bfloat16)


def _mm(A, B):
    M, K = A.shape
    _, N = B.shape
    bm = _pick(M, (1024, 512, 256, 128))
    bn = _pick(N, (1024, 512, 256, 128))
    bk = _pick(K, (1024, 512, 256, 128))
    grid = (M // bm, N // bn, K // bk)
    return pl.pallas_call(
        functools.partial(_mm_body, ksteps=K // bk),
        grid=grid,
        in_specs=[
            pl.BlockSpec((bm, bk), lambda m, n, k: (m, k)),
            pl.BlockSpec((bk, bn), lambda m, n, k: (k, n)),
        ],
        out_specs=pl.BlockSpec((bm, bn), lambda m, n, k: (m, n)),
        out_shape=jax.ShapeDtypeStruct((M, N), jnp.bfloat16),
        scratch_shapes=[pltpu.VMEM((bm, bn), jnp.float32)],
        compiler_params=pltpu.CompilerParams(
            dimension_semantics=("parallel", "parallel", "arbitrary")),
    )(A, B)


def _onehot_body(pm_ref, o_ref, *, bm, bn):
    mi = pl.program_id(0)
    rows = mi * bm + jax.lax.broadcasted_iota(jnp.int32, (bm, bn), 0)
    o_ref[...] = jnp.where(rows == pm_ref[...], 1.0, 0.0).astype(jnp.bfloat16)


def _onehot(perm_p, M):
    N = perm_p.shape[1]
    bm = _pick(M, (1024, 512, 256))
    bn = _pick(N, (512, 256, 128))
    return pl.pallas_call(
        functools.partial(_onehot_body, bm=bm, bn=bn),
        grid=(M // bm, N // bn),
        in_specs=[pl.BlockSpec((1, bn), lambda m, n: (0, n))],
        out_specs=pl.BlockSpec((bm, bn), lambda m, n: (m, n)),
        out_shape=jax.ShapeDtypeStruct((M, N), jnp.bfloat16),
        compiler_params=pltpu.CompilerParams(
            dimension_semantics=("parallel", "parallel")),
    )(perm_p)


def _mm_agg_body(a_ref, hk_ref, hm_ref, w_ref, dk_ref, dm_ref, b_ref,
                 o_ref, acc_ref, *, ksteps, relu):
    ki = pl.program_id(1)

    @pl.when(ki == 0)
    def _():
        acc_ref[...] = jnp.zeros_like(acc_ref)

    u = dk_ref[...] * jnp.dot(hk_ref[...], w_ref[...],
                              preferred_element_type=jnp.float32)
    uh = u.astype(jnp.bfloat16)
    ul = (u - uh.astype(jnp.float32)).astype(jnp.bfloat16)
    a = a_ref[...]
    acc_ref[...] += (jnp.dot(a, uh, preferred_element_type=jnp.float32)
                     + jnp.dot(a, ul, preferred_element_type=jnp.float32))

    @pl.when(ki == ksteps - 1)
    def _():
        um = dm_ref[...] * jnp.dot(hm_ref[...], w_ref[...],
                                   preferred_element_type=jnp.float32)
        out = dm_ref[...] * (acc_ref[...] + 2.0 * um) + b_ref[...]
        if relu:
            out = jnp.maximum(out, 0.0)
        o_ref[...] = out


def _mm_agg(A, h, W, dinv, b, relu):
    M, Cin = h.shape
    C = W.shape[1]
    bm = _pick(M, (1024, 512, 256, 128))
    bk = _pick(M, (1024, 512, 256, 128))
    assert A.shape == (M, M)
    grid = (M // bm, M // bk)
    return pl.pallas_call(
        functools.partial(_mm_agg_body, ksteps=M // bk, relu=relu),
        grid=grid,
        in_specs=[
            pl.BlockSpec((bm, bk), lambda m, k: (m, k)),
            pl.BlockSpec((bk, Cin), lambda m, k: (k, 0)),
            pl.BlockSpec((bm, Cin), lambda m, k: (m, 0)),
            pl.BlockSpec((Cin, C), lambda m, k: (0, 0)),
            pl.BlockSpec((bk, 1), lambda m, k: (k, 0)),
            pl.BlockSpec((bm, 1), lambda m, k: (m, 0)),
            pl.BlockSpec((1, C), lambda m, k: (0, 0)),
        ],
        out_specs=pl.BlockSpec((bm, C), lambda m, k: (m, 0)),
        out_shape=jax.ShapeDtypeStruct((M, C), jnp.float32),
        scratch_shapes=[pltpu.VMEM((bm, C), jnp.float32)],
        compiler_params=pltpu.CompilerParams(
            dimension_semantics=("parallel", "arbitrary")),
    )(A, h, h, W, dinv, dinv, b)


def _rowsum_diag_body(a_ref, rs_ref, dg_ref, acc_ref, *, ksteps, bm, bk):
    mi = pl.program_id(0)
    ki = pl.program_id(1)

    @pl.when(ki == 0)
    def _():
        acc_ref[...] = jnp.zeros_like(acc_ref)

    a = a_ref[...].astype(jnp.float32)
    acc_ref[...] += jnp.sum(a, axis=1, keepdims=True)

    @pl.when(ki == ksteps - 1)
    def _():
        rs_ref[...] = acc_ref[...]

    @pl.when(ki * bk == mi * bm)
    def _():
        rows = jax.lax.broadcasted_iota(jnp.int32, (bm, bk), 0)
        cols = jax.lax.broadcasted_iota(jnp.int32, (bm, bk), 1)
        dg_ref[...] = jnp.sum(jnp.where(rows == cols, a, 0.0), axis=1,
                              keepdims=True)


def _rowsum_diag(A):
    M = A.shape[0]
    bm = _pick(M, (1024, 512, 256, 128))
    bk = bm
    grid = (M // bm, M // bk)
    return pl.pallas_call(
        functools.partial(_rowsum_diag_body, ksteps=M // bk, bm=bm, bk=bk),
        grid=grid,
        in_specs=[pl.BlockSpec((bm, bk), lambda m, k: (m, k))],
        out_specs=[
            pl.BlockSpec((bm, 1), lambda m, k: (m, 0)),
            pl.BlockSpec((bm, 1), lambda m, k: (m, 0)),
        ],
        out_shape=[
            jax.ShapeDtypeStruct((M, 1), jnp.float32),
            jax.ShapeDtypeStruct((M, 1), jnp.float32),
        ],
        scratch_shapes=[pltpu.VMEM((bm, 1), jnp.float32)],
        compiler_params=pltpu.CompilerParams(
            dimension_semantics=("parallel", "arbitrary")),
    )(A)


def _pad_rows(a, rows):
    return jnp.pad(a, ((0, rows - a.shape[0]), (0, 0)))


def _score_topk(h, p, n_valid, k):
    s = jnp.tanh((h[:n_valid] @ p) / jnp.linalg.norm(p))
    _, perm = jax.lax.top_k(s, k)
    return s, perm


def _pool_level(A_p, perm, kp, dself=None):
    k = perm.shape[0]
    G = _pad_rows(A_p[perm], kp)
    perm_p = jnp.full((1, kp), -1, jnp.int32).at[0, :k].set(perm)
    H = _mm(A_p, _onehot(perm_p, A_p.shape[0]))
    Apool = _pad_rows(H[perm], kp)
    if dself is None:
        dr = jnp.zeros((kp, 1), jnp.float32)
        dc = jnp.zeros((1, kp), jnp.float32)
    else:
        dp = jnp.pad(dself[perm], (0, kp - k))
        dr = dp[:, None]
        dc = dp[None, :]
    C, rs = _mm_pool(G, H, Apool, dr, dc)
    dinv = (rs + 2.0) ** -0.5
    return C, dinv


def kernel(x, edge_index, W0, b0, W1, b1, W2, b2, W3, b3,
           p1, p2, p3, U0, ub0, U1, ub1, U2, ub2):
    n = N_NODES
    ei = edge_index.astype(jnp.int32)
    dst, src = ei[1], ei[0]

    A = jnp.zeros((NP, NP), jnp.float32).at[dst, src].add(1.0)
    Ab = A.astype(jnp.bfloat16)
    rs0, dself = _rowsum_diag(Ab)
    dinv0_p = (rs0 + 2.0) ** -0.5
    dself = dself[:, 0]
    x_p = _pad_rows(x, NP)

    def gcn(h_p, A_p, dinv_p, W, b, relu):
        return _mm_agg(A_p, h_p, W, dinv_p, b[None, :], relu)

    h0_p = gcn(x_p, Ab, dinv0_p, W0, b0, True)

    s1, perm1 = _score_topk(h0_p, p1, n, K1)
    A1, dinv1 = _pool_level(Ab, perm1, K1P, dself=dself)
    h1p = _pad_rows(h0_p[perm1] * s1[perm1][:, None], K1P)
    h1 = gcn(h1p, A1, dinv1, W1, b1, True)

    s2, perm2 = _score_topk(h1, p2, K1, K2)
    A2, dinv2 = _pool_level(A1, perm2, K2P)
    h2p = _pad_rows(h1[perm2] * s2[perm2][:, None], K2P)
    h2 = gcn(h2p, A2, dinv2, W2, b2, True)

    s3, perm3 = _score_topk(h2, p3, K2, K3)
    A3, dinv3 = _pool_level(A2, perm3, K3P)
    h3p = _pad_rows(h2[perm3] * s3[perm3][:, None], K3P)
    h = gcn(h3p, A3, dinv3, W3, b3, True)

    h = h2.at[perm3].add(h[:K3])
    h = gcn(h, A2, dinv2, U0, ub0, True)
    h = h1.at[perm2].add(h[:K2])
    h = gcn(h, A1, dinv1, U1, ub1, True)
    h = h0_p.at[perm1].add(h[:K1])
    h = gcn(h, Ab, dinv0_p, U2, ub2, False)
    return h[:n]


# speedup vs baseline: 3.2915x; 1.2956x over previous
import functools
import math

import jax
import jax.numpy as jnp
import numpy as np
from jax.experimental import pallas as pl
from jax.experimental.pallas import tpu as pltpu

N_NODES = 10000
RATIO = 0.5

NP = 10240
K1, K2, K3 = 5000, 2500, 1250
K1P, K2P, K3P = 5120, 2560, 1280


def _pick(M, prefs):
    for p in prefs:
        if M % p == 0:
            return p
    raise ValueError(f"no block size for {M}")


def _mm_pool_body(g_ref, h_ref, ap_ref, dr_ref, dc_ref, o_ref, rs_ref,
                  acc_ref, *, ksteps, bm, bn):
    mi = pl.program_id(0)
    ni = pl.program_id(1)
    ki = pl.program_id(2)

    @pl.when(ki == 0)
    def _():
        acc_ref[...] = jnp.zeros_like(acc_ref)

    acc_ref[...] += jnp.dot(g_ref[...], h_ref[...],
                            preferred_element_type=jnp.float32)

    @pl.when(ki == ksteps - 1)
    def _():
        corr = ap_ref[...].astype(jnp.float32) * (2.0 - dr_ref[...] - dc_ref[...])
        acc = acc_ref[...] + corr
        rows = mi * bm + jax.lax.broadcasted_iota(jnp.int32, (bm, bn), 0)
        cols = ni * bn + jax.lax.broadcasted_iota(jnp.int32, (bm, bn), 1)
        acc = jnp.where(rows == cols, 0.0, acc)
        o_ref[...] = acc.astype(jnp.bfloat16)

        @pl.when(ni == 0)
        def _():
            rs_ref[...] = jnp.zeros_like(rs_ref)

        rs_ref[...] += jnp.sum(acc, axis=1, keepdims=True)


def _mm_pool(G, H, Apool, dr, dc):
    M, K = G.shape
    _, N = H.shape
    bm = _pick(M, (1024, 512, 256, 128))
    bn = _pick(N, (1024, 512, 256, 128))
    bk = _pick(K, (1024, 512, 256, 128))
    grid = (M // bm, N // bn, K // bk)
    return pl.pallas_call(
        functools.partial(_mm_pool_body, ksteps=K // bk, bm=bm, bn=bn),
        grid=grid,
        in_specs=[
            pl.BlockSpec((bm, bk), lambda m, n, k: (m, k)),
            pl.BlockSpec((bk, bn), lambda m, n, k: (k, n)),
            pl.BlockSpec((bm, bn), lambda m, n, k: (m, n)),
            pl.BlockSpec((bm, 1), lambda m, n, k: (m, 0)),
            pl.BlockSpec((1, bn), lambda m, n, k: (0, n)),
        ],
        out_specs=[
            pl.BlockSpec((bm, bn), lambda m, n, k: (m, n)),
            pl.BlockSpec((bm, 1), lambda m, n, k: (m, 0)),
        ],
        out_shape=[
            jax.ShapeDtypeStruct((M, N), jnp.bfloat16),
            jax.ShapeDtypeStruct((M, 1), jnp.float32),
        ],
        scratch_shapes=[pltpu.VMEM((bm, bn), jnp.float32)],
        compiler_params=pltpu.CompilerParams(
            dimension_semantics=("parallel", "parallel", "arbitrary")),
    )(G, H, Apool, dr, dc)


def _tr_body(a_ref, o_ref):
    o_ref[...] = a_ref[...].T


def _transpose(A):
    M, N = A.shape
    bt = _pick(math.gcd(M, N), (1024, 512, 256, 128))
    return pl.pallas_call(
        _tr_body,
        grid=(M // bt, N // bt),
        in_specs=[pl.BlockSpec((bt, bt), lambda i, j: (i, j))],
        out_specs=pl.BlockSpec((bt, bt), lambda i, j: (j, i)),
        out_shape=jax.ShapeDtypeStruct((N, M), A.dtype),
        compiler_params=pltpu.CompilerParams(
            dimension_semantics=("parallel", "parallel")),
    )(A)


def _mm_agg_body(a_ref, hk_ref, hm_ref, w_ref, dk_ref, dm_ref, b_ref,
                 o_ref, acc_ref, *, ksteps, relu):
    ki = pl.program_id(1)

    @pl.when(ki == 0)
    def _():
        acc_ref[...] = jnp.zeros_like(acc_ref)

    u = dk_ref[...] * jnp.dot(hk_ref[...], w_ref[...],
                              preferred_element_type=jnp.float32)
    uh = u.astype(jnp.bfloat16)
    ul = (u - uh.astype(jnp.float32)).astype(jnp.bfloat16)
    a = a_ref[...]
    acc_ref[...] += (jnp.dot(a, uh, preferred_element_type=jnp.float32)
                     + jnp.dot(a, ul, preferred_element_type=jnp.float32))

    @pl.when(ki == ksteps - 1)
    def _():
        um = dm_ref[...] * jnp.dot(hm_ref[...], w_ref[...],
                                   preferred_element_type=jnp.float32)
        out = dm_ref[...] * (acc_ref[...] + 2.0 * um) + b_ref[...]
        if relu:
            out = jnp.maximum(out, 0.0)
        o_ref[...] = out


def _mm_agg(A, h, W, dinv, b, relu):
    M, Cin = h.shape
    C = W.shape[1]
    bm = _pick(M, (1024, 512, 256, 128))
    bk = _pick(M, (1024, 512, 256, 128))
    assert A.shape == (M, M)
    grid = (M // bm, M // bk)
    return pl.pallas_call(
        functools.partial(_mm_agg_body, ksteps=M // bk, relu=relu),
        grid=grid,
        in_specs=[
            pl.BlockSpec((bm, bk), lambda m, k: (m, k)),
            pl.BlockSpec((bk, Cin), lambda m, k: (k, 0)),
            pl.BlockSpec((bm, Cin), lambda m, k: (m, 0)),
            pl.BlockSpec((Cin, C), lambda m, k: (0, 0)),
            pl.BlockSpec((bk, 1), lambda m, k: (k, 0)),
            pl.BlockSpec((bm, 1), lambda m, k: (m, 0)),
            pl.BlockSpec((1, C), lambda m, k: (0, 0)),
        ],
        out_specs=pl.BlockSpec((bm, C), lambda m, k: (m, 0)),
        out_shape=jax.ShapeDtypeStruct((M, C), jnp.float32),
        scratch_shapes=[pltpu.VMEM((bm, C), jnp.float32)],
        compiler_params=pltpu.CompilerParams(
            dimension_semantics=("parallel", "arbitrary")),
    )(A, h, h, W, dinv, dinv, b)


def _rowsum_diag_body(a_ref, rs_ref, dg_ref, acc_ref, *, ksteps, bm, bk):
    mi = pl.program_id(0)
    ki = pl.program_id(1)

    @pl.when(ki == 0)
    def _():
        acc_ref[...] = jnp.zeros_like(acc_ref)

    a = a_ref[...].astype(jnp.float32)
    acc_ref[...] += jnp.sum(a, axis=1, keepdims=True)

    @pl.when(ki == ksteps - 1)
    def _():
        rs_ref[...] = acc_ref[...]

    @pl.when(ki * bk == mi * bm)
    def _():
        rows = jax.lax.broadcasted_iota(jnp.int32, (bm, bk), 0)
        cols = jax.lax.broadcasted_iota(jnp.int32, (bm, bk), 1)
        dg_ref[...] = jnp.sum(jnp.where(rows == cols, a, 0.0), axis=1,
                              keepdims=True)


def _rowsum_diag(A):
    M = A.shape[0]
    bm = _pick(M, (1024, 512, 256, 128))
    bk = bm
    grid = (M // bm, M // bk)
    return pl.pallas_call(
        functools.partial(_rowsum_diag_body, ksteps=M // bk, bm=bm, bk=bk),
        grid=grid,
        in_specs=[pl.BlockSpec((bm, bk), lambda m, k: (m, k))],
        out_specs=[
            pl.BlockSpec((bm, 1), lambda m, k: (m, 0)),
            pl.BlockSpec((bm, 1), lambda m, k: (m, 0)),
        ],
        out_shape=[
            jax.ShapeDtypeStruct((M, 1), jnp.float32),
            jax.ShapeDtypeStruct((M, 1), jnp.float32),
        ],
        scratch_shapes=[pltpu.VMEM((bm, 1), jnp.float32)],
        compiler_params=pltpu.CompilerParams(
            dimension_semantics=("parallel", "arbitrary")),
    )(A)


def _pad_rows(a, rows):
    return jnp.pad(a, ((0, rows - a.shape[0]), (0, 0)))


def _score_topk(h, p, n_valid, k):
    s = jnp.tanh((h[:n_valid] @ p) / jnp.linalg.norm(p))
    _, perm = jax.lax.top_k(s, k)
    return s, perm


def _pool_level(A_p, perm, kp, dself=None):
    k = perm.shape[0]
    np_ = A_p.shape[0]
    G = _pad_rows(A_p[perm], kp)
    AT = _transpose(A_p)
    perm_z = jnp.full((kp,), np_ - 1, jnp.int32).at[:k].set(perm)
    H = _transpose(AT[perm_z])
    Apool = _pad_rows(H[perm], kp)
    if dself is None:
        dr = jnp.zeros((kp, 1), jnp.float32)
        dc = jnp.zeros((1, kp), jnp.float32)
    else:
        dp = jnp.pad(dself[perm], (0, kp - k))
        dr = dp[:, None]
        dc = dp[None, :]
    C, rs = _mm_pool(G, H, Apool, dr, dc)
    dinv = (rs + 2.0) ** -0.5
    return C, dinv


def kernel(x, edge_index, W0, b0, W1, b1, W2, b2, W3, b3,
           p1, p2, p3, U0, ub0, U1, ub1, U2, ub2):
    n = N_NODES
    ei = edge_index.astype(jnp.int32)
    dst, src = ei[1], ei[0]

    A = jnp.zeros((NP, NP), jnp.float32).at[dst, src].add(1.0)
    Ab = A.astype(jnp.bfloat16)
    rs0, dself = _rowsum_diag(Ab)
    dinv0_p = (rs0 + 2.0) ** -0.5
    dself = dself[:, 0]
    x_p = _pad_rows(x, NP)

    def gcn(h_p, A_p, dinv_p, W, b, relu):
        return _mm_agg(A_p, h_p, W, dinv_p, b[None, :], relu)

    h0_p = gcn(x_p, Ab, dinv0_p, W0, b0, True)

    s1, perm1 = _score_topk(h0_p, p1, n, K1)
    A1, dinv1 = _pool_level(Ab, perm1, K1P, dself=dself)
    h1p = _pad_rows(h0_p[perm1] * s1[perm1][:, None], K1P)
    h1 = gcn(h1p, A1, dinv1, W1, b1, True)

    s2, perm2 = _score_topk(h1, p2, K1, K2)
    A2, dinv2 = _pool_level(A1, perm2, K2P)
    h2p = _pad_rows(h1[perm2] * s2[perm2][:, None], K2P)
    h2 = gcn(h2p, A2, dinv2, W2, b2, True)

    s3, perm3 = _score_topk(h2, p3, K2, K3)
    A3, dinv3 = _pool_level(A2, perm3, K3P)
    h3p = _pad_rows(h2[perm3] * s3[perm3][:, None], K3P)
    h = gcn(h3p, A3, dinv3, W3, b3, True)

    h = h2.at[perm3].add(h[:K3])
    h = gcn(h, A2, dinv2, U0, ub0, True)
    h = h1.at[perm2].add(h[:K2])
    h = gcn(h, A1, dinv1, U1, ub1, True)
    h = h0_p.at[perm1].add(h[:K1])
    h = gcn(h, Ab, dinv0_p, U2, ub2, False)
    return h[:n]
